# baseline (speedup 1.0000x reference)
import functools

import jax
import jax.numpy as jnp
from jax import lax
from jax.experimental import pallas as pl
from jax.experimental.pallas import tpu as pltpu
from jax.experimental.pallas import tpu_sc as plsc

B, S, H, E, V = 32, 128, 1024, 256, 8192


def _build_sc_gather(n_rows: int):
    info = plsc.get_sparse_core_info()
    nc, ns = info.num_cores, info.num_subcores
    nw = nc * ns
    assert n_rows % (8 * nw) == 0
    rows_per_w = n_rows // nw

    @functools.partial(
        pl.kernel,
        out_type=jax.ShapeDtypeStruct((n_rows, E), jnp.float32),
        mesh=plsc.VectorSubcoreMesh(core_axis_name="c", subcore_axis_name="s"),
        scratch_types=[
            pltpu.VMEM((rows_per_w,), jnp.int32),
            pltpu.VMEM((rows_per_w, E), jnp.float32),
            pltpu.SemaphoreType.DMA,
        ],
    )
    def gather(table_hbm, idx_hbm, out_hbm, idx_v, rows_v, sem):
        wid = lax.axis_index("s") * nc + lax.axis_index("c")
        base = wid * rows_per_w
        pltpu.sync_copy(idx_hbm.at[pl.ds(base, rows_per_w)], idx_v)
        pltpu.async_copy(table_hbm.at[idx_v], rows_v, sem).wait()
        pltpu.sync_copy(rows_v, out_hbm.at[pl.ds(base, rows_per_w)])

    return gather


def _gi_body(x_ref, w_ref, b_ref, o_ref):
    x = x_ref[...].astype(jnp.bfloat16)
    o_ref[...] = (
        jnp.dot(x, w_ref[...], preferred_element_type=jnp.float32) + b_ref[...]
    ).astype(jnp.bfloat16)


def _input_proj(emb, w_ihT, b_ih2, interpret=False):
    m = emb.shape[0]
    m_blk = 512
    grid = (m // m_blk,)
    return pl.pallas_call(
        _gi_body,
        grid=grid,
        in_specs=[
            pl.BlockSpec((m_blk, E), lambda i: (i, 0)),
            pl.BlockSpec((E, 3 * H), lambda i: (0, 0)),
            pl.BlockSpec((1, 3 * H), lambda i: (0, 0)),
        ],
        out_specs=pl.BlockSpec((m_blk, 3 * H), lambda i: (i, 0)),
        out_shape=jax.ShapeDtypeStruct((m, 3 * H), jnp.bfloat16),
        interpret=interpret,
    )(emb, w_ihT, b_ih2)


_T_BLK = 8


def _gru_body(gi_ref, w_ref, b_ref, o_ref, h_ref):
    t = pl.program_id(0)

    @pl.when(t == 0)
    def _():
        h_ref[...] = jnp.zeros_like(h_ref)

    h = h_ref[...]
    w = w_ref[...]
    b = b_ref[...]
    for k in range(_T_BLK):
        gh = (
            jnp.dot(
                h.astype(jnp.bfloat16), w, preferred_element_type=jnp.float32
            )
            + b
        )
        gi = gi_ref[k]
        r = jax.nn.sigmoid(gi[:, :H] + gh[:, :H])
        z = jax.nn.sigmoid(gi[:, H : 2 * H] + gh[:, H : 2 * H])
        n = jnp.tanh(gi[:, 2 * H :] + r * gh[:, 2 * H :])
        h = (1.0 - z) * n + z * h
        o_ref[k] = h.astype(jnp.bfloat16)
    h_ref[...] = h


def _gru(gi_all, w_hhT, b_hh2, interpret=False):
    return pl.pallas_call(
        _gru_body,
        grid=(S // _T_BLK,),
        in_specs=[
            pl.BlockSpec((_T_BLK, B, 3 * H), lambda t: (t, 0, 0)),
            pl.BlockSpec((H, 3 * H), lambda t: (0, 0)),
            pl.BlockSpec((1, 3 * H), lambda t: (0, 0)),
        ],
        out_specs=pl.BlockSpec((_T_BLK, B, H), lambda t: (t, 0, 0)),
        out_shape=jax.ShapeDtypeStruct((S, B, H), jnp.bfloat16),
        scratch_shapes=[pltpu.VMEM((B, H), jnp.float32)],
        compiler_params=pltpu.CompilerParams(
            dimension_semantics=("arbitrary",)
        ),
        interpret=interpret,
    )(gi_all, w_hhT, b_hh2)


_S_BLK = 8
_N_BLK = 2048


def _fc_body(x_ref, w_ref, b_ref, o_ref):
    x = x_ref[...]
    xt = jnp.swapaxes(x, 0, 1).reshape(B * _S_BLK, H)
    y = (
        lax.dot_general(
            xt, w_ref[...], (((1,), (1,)), ((), ())),
            preferred_element_type=jnp.float32,
        )
        + b_ref[...]
    )
    o_ref[...] = y.reshape(B, _S_BLK, _N_BLK)


def _fc(outs, w_fc, b_fc2, interpret=False):
    grid = (V // _N_BLK, S // _S_BLK)
    return pl.pallas_call(
        _fc_body,
        grid=grid,
        in_specs=[
            pl.BlockSpec((_S_BLK, B, H), lambda n, s: (s, 0, 0)),
            pl.BlockSpec((_N_BLK, H), lambda n, s: (n, 0)),
            pl.BlockSpec((1, _N_BLK), lambda n, s: (0, n)),
        ],
        out_specs=pl.BlockSpec((B, _S_BLK, _N_BLK), lambda n, s: (0, s, n)),
        out_shape=jax.ShapeDtypeStruct((B, S, V), jnp.float32),
        compiler_params=pltpu.CompilerParams(
            dimension_semantics=("parallel", "arbitrary")
        ),
        interpret=interpret,
    )(outs, w_fc, b_fc2)


def kernel(embed_table, W_ih, W_hh, b_ih, b_hh, W_fc, b_fc, y_inp):
    idx = jnp.transpose(y_inp).reshape(-1).astype(jnp.int32)

    emb = _build_sc_gather(S * B)(embed_table, idx)

    bf = jnp.bfloat16
    gi = _input_proj(
        emb, jnp.transpose(W_ih).astype(bf), b_ih.reshape(1, 3 * H)
    )
    outs = _gru(
        gi.reshape(S, B, 3 * H),
        jnp.transpose(W_hh).astype(bf),
        b_hh.reshape(1, 3 * H),
    )
    logits = _fc(outs, W_fc.astype(bf), b_fc.reshape(1, V))
    h_last = outs[S - 1][None].astype(jnp.float32)
    return (logits, h_last)


# speedup vs baseline: 1.0836x; 1.0836x over previous
import functools

import jax
import jax.numpy as jnp
from jax import lax
from jax.experimental import pallas as pl
from jax.experimental.pallas import tpu as pltpu
from jax.experimental.pallas import tpu_sc as plsc

B, S, H, E, V = 32, 128, 1024, 256, 8192


def _build_sc_gather(n_rows: int):
    info = plsc.get_sparse_core_info()
    nc, ns = info.num_cores, info.num_subcores
    nw = nc * ns
    assert n_rows % (8 * nw) == 0
    rows_per_w = n_rows // nw

    @functools.partial(
        pl.kernel,
        out_type=jax.ShapeDtypeStruct((n_rows, E), jnp.float32),
        mesh=plsc.VectorSubcoreMesh(core_axis_name="c", subcore_axis_name="s"),
        scratch_types=[
            pltpu.VMEM((rows_per_w,), jnp.int32),
            pltpu.VMEM((rows_per_w, E), jnp.float32),
            pltpu.SemaphoreType.DMA,
        ],
    )
    def gather(table_hbm, idx_hbm, out_hbm, idx_v, rows_v, sem):
        wid = lax.axis_index("s") * nc + lax.axis_index("c")
        base = wid * rows_per_w
        pltpu.sync_copy(idx_hbm.at[pl.ds(base, rows_per_w)], idx_v)
        pltpu.async_copy(table_hbm.at[idx_v], rows_v, sem).wait()
        pltpu.sync_copy(rows_v, out_hbm.at[pl.ds(base, rows_per_w)])

    return gather


def _gi_body(x_ref, w_ref, b_ref, o_ref):
    x = x_ref[...].astype(jnp.bfloat16)
    o_ref[...] = (
        jnp.dot(x, w_ref[...], preferred_element_type=jnp.float32) + b_ref[...]
    ).astype(jnp.bfloat16)


def _input_proj(emb, w_ihT, b_ih2, interpret=False):
    m = emb.shape[0]
    m_blk = 512
    grid = (m // m_blk,)
    return pl.pallas_call(
        _gi_body,
        grid=grid,
        in_specs=[
            pl.BlockSpec((m_blk, E), lambda i: (i, 0)),
            pl.BlockSpec((E, 3 * H), lambda i: (0, 0)),
            pl.BlockSpec((1, 3 * H), lambda i: (0, 0)),
        ],
        out_specs=pl.BlockSpec((m_blk, 3 * H), lambda i: (i, 0)),
        out_shape=jax.ShapeDtypeStruct((m, 3 * H), jnp.bfloat16),
        interpret=interpret,
    )(emb, w_ihT, b_ih2)


_T_BLK = 8


def _gru_body(gi_ref, w_ref, b_ref, o_ref, h_ref):
    t = pl.program_id(0)

    @pl.when(t == 0)
    def _():
        h_ref[...] = jnp.zeros_like(h_ref)

    h = h_ref[...]
    w = w_ref[...]
    b = b_ref[...]
    for k in range(_T_BLK):
        gh = (
            jnp.dot(
                h.astype(jnp.bfloat16), w, preferred_element_type=jnp.float32
            )
            + b
        )
        gi = gi_ref[k]
        r = jax.nn.sigmoid(gi[:, :H] + gh[:, :H])
        z = jax.nn.sigmoid(gi[:, H : 2 * H] + gh[:, H : 2 * H])
        n = jnp.tanh(gi[:, 2 * H :] + r * gh[:, 2 * H :])
        h = (1.0 - z) * n + z * h
        o_ref[k] = h.astype(jnp.bfloat16)
    h_ref[...] = h


def _gru(gi_all, w_hhT, b_hh2, interpret=False):
    return pl.pallas_call(
        _gru_body,
        grid=(S // _T_BLK,),
        in_specs=[
            pl.BlockSpec((_T_BLK, B, 3 * H), lambda t: (t, 0, 0)),
            pl.BlockSpec((H, 3 * H), lambda t: (0, 0)),
            pl.BlockSpec((1, 3 * H), lambda t: (0, 0)),
        ],
        out_specs=pl.BlockSpec((_T_BLK, B, H), lambda t: (t, 0, 0)),
        out_shape=jax.ShapeDtypeStruct((S, B, H), jnp.bfloat16),
        scratch_shapes=[pltpu.VMEM((B, H), jnp.float32)],
        compiler_params=pltpu.CompilerParams(
            dimension_semantics=("arbitrary",)
        ),
        interpret=interpret,
    )(gi_all, w_hhT, b_hh2)


_S_BLK = 8
_N_BLK = 8192


def _fc_body(x_ref, w_ref, b_ref, o_ref):
    x = x_ref[...]
    xt = jnp.swapaxes(x, 0, 1).reshape(B * _S_BLK, H)
    y = (
        lax.dot_general(
            xt, w_ref[...], (((1,), (1,)), ((), ())),
            preferred_element_type=jnp.float32,
        )
        + b_ref[...]
    )
    o_ref[...] = y.reshape(B, _S_BLK, _N_BLK)


def _fc(outs, w_fc, b_fc2, interpret=False):
    grid = (V // _N_BLK, S // _S_BLK)
    return pl.pallas_call(
        _fc_body,
        grid=grid,
        in_specs=[
            pl.BlockSpec((_S_BLK, B, H), lambda n, s: (s, 0, 0)),
            pl.BlockSpec((_N_BLK, H), lambda n, s: (n, 0)),
            pl.BlockSpec((1, _N_BLK), lambda n, s: (0, n)),
        ],
        out_specs=pl.BlockSpec((B, _S_BLK, _N_BLK), lambda n, s: (0, s, n)),
        out_shape=jax.ShapeDtypeStruct((B, S, V), jnp.float32),
        compiler_params=pltpu.CompilerParams(
            dimension_semantics=("parallel", "arbitrary")
        ),
        interpret=interpret,
    )(outs, w_fc, b_fc2)


def kernel(embed_table, W_ih, W_hh, b_ih, b_hh, W_fc, b_fc, y_inp):
    idx = jnp.transpose(y_inp).reshape(-1).astype(jnp.int32)

    emb = _build_sc_gather(S * B)(embed_table, idx)

    bf = jnp.bfloat16
    gi = _input_proj(
        emb, jnp.transpose(W_ih).astype(bf), b_ih.reshape(1, 3 * H)
    )
    outs = _gru(
        gi.reshape(S, B, 3 * H),
        jnp.transpose(W_hh).astype(bf),
        b_hh.reshape(1, 3 * H),
    )
    logits = _fc(outs, W_fc.astype(bf), b_fc.reshape(1, V))
    h_last = outs[S - 1][None].astype(jnp.float32)
    return (logits, h_last)


# speedup vs baseline: 1.0846x; 1.0009x over previous
import functools

import jax
import jax.numpy as jnp
from jax import lax
from jax.experimental import pallas as pl
from jax.experimental.pallas import tpu as pltpu
from jax.experimental.pallas import tpu_sc as plsc

B, S, H, E, V = 32, 128, 1024, 256, 8192


def _build_sc_gather(n_rows: int):
    info = plsc.get_sparse_core_info()
    nc, ns = info.num_cores, info.num_subcores
    nw = nc * ns
    assert n_rows % (8 * nw) == 0
    rows_per_w = n_rows // nw

    @functools.partial(
        pl.kernel,
        out_type=jax.ShapeDtypeStruct((n_rows, E), jnp.float32),
        mesh=plsc.VectorSubcoreMesh(core_axis_name="c", subcore_axis_name="s"),
        scratch_types=[
            pltpu.VMEM((rows_per_w,), jnp.int32),
            pltpu.VMEM((rows_per_w, E), jnp.float32),
            pltpu.SemaphoreType.DMA,
        ],
    )
    def gather(table_hbm, idx_hbm, out_hbm, idx_v, rows_v, sem):
        wid = lax.axis_index("s") * nc + lax.axis_index("c")
        base = wid * rows_per_w
        pltpu.sync_copy(idx_hbm.at[pl.ds(base, rows_per_w)], idx_v)
        pltpu.async_copy(table_hbm.at[idx_v], rows_v, sem).wait()
        pltpu.sync_copy(rows_v, out_hbm.at[pl.ds(base, rows_per_w)])

    return gather


def _gi_body(x_ref, w_ref, b_ref, o_ref):
    x = x_ref[...].astype(jnp.bfloat16)
    o_ref[...] = (
        jnp.dot(x, w_ref[...], preferred_element_type=jnp.float32) + b_ref[...]
    ).astype(jnp.bfloat16)


def _input_proj(emb, w_ihT, b_ih2, interpret=False):
    m = emb.shape[0]
    m_blk = 512
    grid = (m // m_blk,)
    return pl.pallas_call(
        _gi_body,
        grid=grid,
        in_specs=[
            pl.BlockSpec((m_blk, E), lambda i: (i, 0)),
            pl.BlockSpec((E, 3 * H), lambda i: (0, 0)),
            pl.BlockSpec((1, 3 * H), lambda i: (0, 0)),
        ],
        out_specs=pl.BlockSpec((m_blk, 3 * H), lambda i: (i, 0)),
        out_shape=jax.ShapeDtypeStruct((m, 3 * H), jnp.bfloat16),
        interpret=interpret,
    )(emb, w_ihT, b_ih2)


_T_BLK = 8


def _gru_body(gi_ref, w_ref, b_ref, o_ref, h_ref):
    t = pl.program_id(0)

    @pl.when(t == 0)
    def _():
        h_ref[...] = jnp.zeros_like(h_ref)

    h = h_ref[...]
    w = w_ref[...]
    b = b_ref[...]
    for k in range(_T_BLK):
        gh = (
            jnp.dot(
                h.astype(jnp.bfloat16), w, preferred_element_type=jnp.float32
            )
            + b
        )
        gi = gi_ref[k]
        r = jax.nn.sigmoid(gi[:, :H] + gh[:, :H])
        z = jax.nn.sigmoid(gi[:, H : 2 * H] + gh[:, H : 2 * H])
        n = jnp.tanh(gi[:, 2 * H :] + r * gh[:, 2 * H :])
        h = (1.0 - z) * n + z * h
        o_ref[k] = h.astype(jnp.bfloat16)
    h_ref[...] = h


def _gru(gi_all, w_hhT, b_hh2, interpret=False):
    return pl.pallas_call(
        _gru_body,
        grid=(S // _T_BLK,),
        in_specs=[
            pl.BlockSpec((_T_BLK, B, 3 * H), lambda t: (t, 0, 0)),
            pl.BlockSpec((H, 3 * H), lambda t: (0, 0)),
            pl.BlockSpec((1, 3 * H), lambda t: (0, 0)),
        ],
        out_specs=pl.BlockSpec((_T_BLK, B, H), lambda t: (t, 0, 0)),
        out_shape=jax.ShapeDtypeStruct((S, B, H), jnp.bfloat16),
        scratch_shapes=[pltpu.VMEM((B, H), jnp.float32)],
        compiler_params=pltpu.CompilerParams(
            dimension_semantics=("arbitrary",)
        ),
        interpret=interpret,
    )(gi_all, w_hhT, b_hh2)


_S_BLK = 16
_N_BLK = 8192


def _fc_body(x_ref, w_ref, b_ref, o_ref):
    x = x_ref[...]
    xt = jnp.swapaxes(x, 0, 1).reshape(B * _S_BLK, H)
    y = (
        lax.dot_general(
            xt, w_ref[...], (((1,), (1,)), ((), ())),
            preferred_element_type=jnp.float32,
        )
        + b_ref[...]
    )
    o_ref[...] = y.reshape(B, _S_BLK, _N_BLK)


def _fc(outs, w_fc, b_fc2, interpret=False):
    grid = (V // _N_BLK, S // _S_BLK)
    return pl.pallas_call(
        _fc_body,
        grid=grid,
        in_specs=[
            pl.BlockSpec((_S_BLK, B, H), lambda n, s: (s, 0, 0)),
            pl.BlockSpec((_N_BLK, H), lambda n, s: (n, 0)),
            pl.BlockSpec((1, _N_BLK), lambda n, s: (0, n)),
        ],
        out_specs=pl.BlockSpec((B, _S_BLK, _N_BLK), lambda n, s: (0, s, n)),
        out_shape=jax.ShapeDtypeStruct((B, S, V), jnp.float32),
        compiler_params=pltpu.CompilerParams(
            dimension_semantics=("parallel", "arbitrary")
        ),
        interpret=interpret,
    )(outs, w_fc, b_fc2)


def kernel(embed_table, W_ih, W_hh, b_ih, b_hh, W_fc, b_fc, y_inp):
    idx = jnp.transpose(y_inp).reshape(-1).astype(jnp.int32)

    emb = _build_sc_gather(S * B)(embed_table, idx)

    bf = jnp.bfloat16
    gi = _input_proj(
        emb, jnp.transpose(W_ih).astype(bf), b_ih.reshape(1, 3 * H)
    )
    outs = _gru(
        gi.reshape(S, B, 3 * H),
        jnp.transpose(W_hh).astype(bf),
        b_hh.reshape(1, 3 * H),
    )
    logits = _fc(outs, W_fc.astype(bf), b_fc.reshape(1, V))
    h_last = outs[S - 1][None].astype(jnp.float32)
    return (logits, h_last)


# speedup vs baseline: 1.0992x; 1.0135x over previous
import functools

import jax
import jax.numpy as jnp
from jax import lax
from jax.experimental import pallas as pl
from jax.experimental.pallas import tpu as pltpu
from jax.experimental.pallas import tpu_sc as plsc

B, S, H, E, V = 32, 128, 1024, 256, 8192


def _build_sc_gather(n_rows: int):
    info = plsc.get_sparse_core_info()
    nc, ns = info.num_cores, info.num_subcores
    nw = nc * ns
    assert n_rows % (8 * nw) == 0
    rows_per_w = n_rows // nw

    @functools.partial(
        pl.kernel,
        out_type=jax.ShapeDtypeStruct((n_rows, E), jnp.float32),
        mesh=plsc.VectorSubcoreMesh(core_axis_name="c", subcore_axis_name="s"),
        scratch_types=[
            pltpu.VMEM((rows_per_w,), jnp.int32),
            pltpu.VMEM((rows_per_w, E), jnp.float32),
            pltpu.SemaphoreType.DMA,
        ],
    )
    def gather(table_hbm, idx_hbm, out_hbm, idx_v, rows_v, sem):
        wid = lax.axis_index("s") * nc + lax.axis_index("c")
        base = wid * rows_per_w
        pltpu.sync_copy(idx_hbm.at[pl.ds(base, rows_per_w)], idx_v)
        pltpu.async_copy(table_hbm.at[idx_v], rows_v, sem).wait()
        pltpu.sync_copy(rows_v, out_hbm.at[pl.ds(base, rows_per_w)])

    return gather


def _gi_body(x_ref, w_ref, b_ref, wfc_ref, o_ref, wfcb_ref):
    x = x_ref[...].astype(jnp.bfloat16)
    o_ref[...] = (
        jnp.dot(x, w_ref[...], preferred_element_type=jnp.float32) + b_ref[...]
    ).astype(jnp.bfloat16)
    wfcb_ref[...] = wfc_ref[...].astype(jnp.bfloat16)


def _input_proj(emb, w_ihT, b_ih2, w_fc, interpret=False):
    m = emb.shape[0]
    m_blk = 512
    grid = (m // m_blk,)
    v_blk = V // (m // m_blk)
    return pl.pallas_call(
        _gi_body,
        grid=grid,
        in_specs=[
            pl.BlockSpec((m_blk, E), lambda i: (i, 0)),
            pl.BlockSpec((E, 3 * H), lambda i: (0, 0)),
            pl.BlockSpec((1, 3 * H), lambda i: (0, 0)),
            pl.BlockSpec((v_blk, H), lambda i: (i, 0)),
        ],
        out_specs=[
            pl.BlockSpec((m_blk, 3 * H), lambda i: (i, 0)),
            pl.BlockSpec((v_blk, H), lambda i: (i, 0)),
        ],
        out_shape=[
            jax.ShapeDtypeStruct((m, 3 * H), jnp.bfloat16),
            jax.ShapeDtypeStruct((V, H), jnp.bfloat16),
        ],
        interpret=interpret,
    )(emb, w_ihT, b_ih2, w_fc)


_T_BLK = 8


def _gru_body(gi_ref, w_ref, b_ref, o_ref, h_ref):
    t = pl.program_id(0)

    @pl.when(t == 0)
    def _():
        h_ref[...] = jnp.zeros_like(h_ref)

    h = h_ref[...]
    w = w_ref[...]
    b = b_ref[...]
    for k in range(_T_BLK):
        gh = (
            jnp.dot(
                h.astype(jnp.bfloat16), w, preferred_element_type=jnp.float32
            )
            + b
        )
        gi = gi_ref[k]
        r = jax.nn.sigmoid(gi[:, :H] + gh[:, :H])
        z = jax.nn.sigmoid(gi[:, H : 2 * H] + gh[:, H : 2 * H])
        n = jnp.tanh(gi[:, 2 * H :] + r * gh[:, 2 * H :])
        h = (1.0 - z) * n + z * h
        o_ref[:, k, :] = h.astype(jnp.bfloat16)
    h_ref[...] = h


def _gru(gi_all, w_hhT, b_hh2, interpret=False):
    return pl.pallas_call(
        _gru_body,
        grid=(S // _T_BLK,),
        in_specs=[
            pl.BlockSpec((_T_BLK, B, 3 * H), lambda t: (t, 0, 0)),
            pl.BlockSpec((H, 3 * H), lambda t: (0, 0)),
            pl.BlockSpec((1, 3 * H), lambda t: (0, 0)),
        ],
        out_specs=pl.BlockSpec((B, _T_BLK, H), lambda t: (0, t, 0)),
        out_shape=jax.ShapeDtypeStruct((B, S, H), jnp.bfloat16),
        scratch_shapes=[pltpu.VMEM((B, H), jnp.float32)],
        compiler_params=pltpu.CompilerParams(
            dimension_semantics=("arbitrary",)
        ),
        interpret=interpret,
    )(gi_all, w_hhT, b_hh2)


_S_BLK = 16
_N_BLK = 8192


_B_BLK = 2


def _fc_body(x_ref, w_ref, b_ref, o_ref):
    xt = x_ref[...].reshape(_B_BLK * S, H)
    y = (
        lax.dot_general(
            xt, w_ref[...], (((1,), (1,)), ((), ())),
            preferred_element_type=jnp.float32,
        )
        + b_ref[...]
    )
    o_ref[...] = y.reshape(_B_BLK, S, _N_BLK)


def _fc(outs, w_fc, b_fc2, interpret=False):
    grid = (V // _N_BLK, B // _B_BLK)
    return pl.pallas_call(
        _fc_body,
        grid=grid,
        in_specs=[
            pl.BlockSpec((_B_BLK, S, H), lambda n, m: (m, 0, 0)),
            pl.BlockSpec((_N_BLK, H), lambda n, m: (n, 0)),
            pl.BlockSpec((1, _N_BLK), lambda n, m: (0, n)),
        ],
        out_specs=pl.BlockSpec((_B_BLK, S, _N_BLK), lambda n, m: (m, 0, n)),
        out_shape=jax.ShapeDtypeStruct((B, S, V), jnp.float32),
        compiler_params=pltpu.CompilerParams(
            dimension_semantics=("parallel", "arbitrary")
        ),
        interpret=interpret,
    )(outs, w_fc, b_fc2)


def kernel(embed_table, W_ih, W_hh, b_ih, b_hh, W_fc, b_fc, y_inp):
    idx = jnp.transpose(y_inp).reshape(-1).astype(jnp.int32)

    emb = _build_sc_gather(S * B)(embed_table, idx)

    bf = jnp.bfloat16
    gi, w_fc_bf = _input_proj(
        emb, jnp.transpose(W_ih).astype(bf), b_ih.reshape(1, 3 * H), W_fc
    )
    outs = _gru(
        gi.reshape(S, B, 3 * H),
        jnp.transpose(W_hh).astype(bf),
        b_hh.reshape(1, 3 * H),
    )
    logits = _fc(outs, w_fc_bf, b_fc.reshape(1, V))
    h_last = outs[:, S - 1][None].astype(jnp.float32)
    return (logits, h_last)
